# baseline (speedup 1.0000x reference)
import functools

import jax
import jax.numpy as jnp
from jax import lax
from jax.experimental import pallas as pl
from jax.experimental.pallas import tpu as pltpu
from jax.experimental.pallas import tpu_sc as plsc

B = 4
S = 4096
D = 2048
NC = 2
NS = 16
NW = NC * NS
S_PER_W = S // NW
C = 16
NCHUNK = S_PER_W // C
NITEMS = NCHUNK * B

_MESH = plsc.VectorSubcoreMesh(core_axis_name="c", subcore_axis_name="s")


@functools.partial(
    pl.kernel,
    out_type=jax.ShapeDtypeStruct((B * S, D), jnp.float32),
    mesh=_MESH,
    scratch_types=[
        pltpu.VMEM((C,), jnp.int32),
        pltpu.VMEM((C,), jnp.int32),
        pltpu.VMEM((C, D), jnp.float32),
        pltpu.VMEM((C, D), jnp.float32),
        pltpu.VMEM((C, D), jnp.float32),
        pltpu.SemaphoreType.DMA,
        pltpu.SemaphoreType.DMA,
        pltpu.SemaphoreType.DMA,
        pltpu.SemaphoreType.DMA,
    ],
)
def _embed(x_hbm, word_hbm, pos_hbm, out_hbm,
           idx0, idx1, pos_v, acc0, acc1, g0, g1, w0, w1):
    sid = lax.axis_index("s")
    wid = sid * NC + lax.axis_index("c")
    s0 = wid * S_PER_W
    idx = (idx0, idx1)
    acc = (acc0, acc1)
    gsem = (g0, g1)
    wsem = (w0, w1)

    def flat_base(k):
        ci, b = divmod(k, B)
        return b * S + s0 + ci * C

    def fire_gather(k, slot):
        pltpu.sync_copy(x_hbm.at[pl.ds(flat_base(k), C)], idx[slot])
        pltpu.async_copy(word_hbm.at[idx[slot]], acc[slot], gsem[slot])

    def wait_gather(slot):
        pltpu.make_async_copy(word_hbm.at[idx[slot]], acc[slot],
                              gsem[slot]).wait()

    def fire_write(k, slot):
        pltpu.async_copy(acc[slot], out_hbm.at[pl.ds(flat_base(k), C)],
                         wsem[slot])

    def wait_write(k, slot):
        pltpu.make_async_copy(acc[slot], out_hbm.at[pl.ds(flat_base(k), C)],
                              wsem[slot]).wait()

    fire_gather(0, 0)

    for k in range(NITEMS):
        ci, b = divmod(k, B)
        slot = k % 2
        if b == 0:
            pltpu.sync_copy(pos_hbm.at[pl.ds(s0 + ci * C, C)], pos_v)
        if k + 1 < NITEMS:
            if k >= 1:
                wait_write(k - 1, 1 - slot)
            fire_gather(k + 1, 1 - slot)
        wait_gather(slot)

        @pl.loop(0, C)
        def _row(r):
            @pl.loop(0, D, step=16, unroll=8)
            def _col(c0):
                sl = pl.ds(c0, 16)
                acc[slot][r, sl] = acc[slot][r, sl] + pos_v[r, sl]

        fire_write(k, slot)

    wait_write(NITEMS - 2, NITEMS % 2)
    wait_write(NITEMS - 1, 1 - NITEMS % 2)


def kernel(x, word_table, pos_table):
    out = _embed(x.reshape(B * S), word_table, pos_table)
    return out.reshape(B, S, D)


# speedup vs baseline: 2.2962x; 2.2962x over previous
import functools

import jax
import jax.numpy as jnp
from jax import lax
from jax.experimental import pallas as pl
from jax.experimental.pallas import tpu as pltpu
from jax.experimental.pallas import tpu_sc as plsc

B = 4
S = 4096
D = 2048
NC = 2
NS = 16
NW = NC * NS
S_PER_W = S // NW
C = 16
NCHUNK = S_PER_W // C
NITEMS = NCHUNK * B

_MESH = plsc.VectorSubcoreMesh(core_axis_name="c", subcore_axis_name="s")


@functools.partial(
    pl.kernel,
    out_type=jax.ShapeDtypeStruct((B * S, D), jnp.float32),
    mesh=_MESH,
    scratch_types=[
        pltpu.VMEM((C,), jnp.int32),
        pltpu.VMEM((C,), jnp.int32),
        pltpu.VMEM((C, D), jnp.float32),
        pltpu.VMEM((C, D), jnp.float32),
        pltpu.VMEM((C, D), jnp.float32),
        pltpu.SemaphoreType.DMA,
        pltpu.SemaphoreType.DMA,
        pltpu.SemaphoreType.DMA,
        pltpu.SemaphoreType.DMA,
    ],
)
def _embed(x_hbm, word_hbm, pos_hbm, out_hbm,
           idx0, idx1, pos_v, acc0, acc1, g0, g1, w0, w1):
    sid = lax.axis_index("s")
    wid = sid * NC + lax.axis_index("c")
    s0 = wid * S_PER_W
    idx = (idx0, idx1)
    acc = (acc0, acc1)
    gsem = (g0, g1)
    wsem = (w0, w1)

    def flat_base(k):
        ci, b = divmod(k, B)
        return b * S + s0 + ci * C

    def fire_gather(k, slot):
        pltpu.sync_copy(x_hbm.at[pl.ds(flat_base(k), C)], idx[slot])
        pltpu.async_copy(word_hbm.at[idx[slot]], acc[slot], gsem[slot])

    def wait_gather(slot):
        pltpu.make_async_copy(word_hbm.at[idx[slot]], acc[slot],
                              gsem[slot]).wait()

    def fire_write(k, slot):
        pltpu.async_copy(acc[slot], out_hbm.at[pl.ds(flat_base(k), C)],
                         wsem[slot])

    def wait_write(k, slot):
        pltpu.make_async_copy(acc[slot], out_hbm.at[pl.ds(flat_base(k), C)],
                              wsem[slot]).wait()

    fire_gather(0, 0)

    for k in range(NITEMS):
        ci, b = divmod(k, B)
        slot = k % 2
        if b == 0:
            pltpu.sync_copy(pos_hbm.at[pl.ds(s0 + ci * C, C)], pos_v)
        if k + 1 < NITEMS:
            if k >= 1:
                wait_write(k - 1, 1 - slot)
            fire_gather(k + 1, 1 - slot)
        wait_gather(slot)

        @pl.loop(0, C)
        def _row(r):
            @plsc.parallel_loop(0, D, 16, unroll=8)
            def _col(c0):
                plsc.addupdate(
                    acc[slot].at[r, pl.ds(c0, 16)], pos_v[r, pl.ds(c0, 16)]
                )

        fire_write(k, slot)

    wait_write(NITEMS - 2, NITEMS % 2)
    wait_write(NITEMS - 1, 1 - NITEMS % 2)


def kernel(x, word_table, pos_table):
    out = _embed(x.reshape(B * S), word_table, pos_table)
    return out.reshape(B, S, D)


# speedup vs baseline: 2.4816x; 1.0808x over previous
import functools

import jax
import jax.numpy as jnp
from jax import lax
from jax.experimental import pallas as pl
from jax.experimental.pallas import tpu as pltpu
from jax.experimental.pallas import tpu_sc as plsc

B = 4
S = 4096
D = 2048
NC = 2
NS = 16
NW = NC * NS
S_PER_W = S // NW
C = 16
NCHUNK = S_PER_W // C
NITEMS = NCHUNK * B

_MESH = plsc.VectorSubcoreMesh(core_axis_name="c", subcore_axis_name="s")


@functools.partial(
    pl.kernel,
    out_type=jax.ShapeDtypeStruct((B * S, D), jnp.float32),
    mesh=_MESH,
    scratch_types=[
        pltpu.VMEM((B, S_PER_W), jnp.int32),
        pltpu.VMEM((C, D), jnp.float32),
        pltpu.VMEM((C, D), jnp.float32),
        pltpu.VMEM((C, D), jnp.float32),
        pltpu.SemaphoreType.DMA,
        pltpu.SemaphoreType.DMA,
        pltpu.SemaphoreType.DMA,
        pltpu.SemaphoreType.DMA,
    ],
)
def _embed(x_hbm, word_hbm, pos_hbm, out_hbm,
           idx_all, pos_v, acc0, acc1, g0, g1, w0, w1):
    sid = lax.axis_index("s")
    wid = sid * NC + lax.axis_index("c")
    s0 = wid * S_PER_W
    acc = (acc0, acc1)
    gsem = (g0, g1)
    wsem = (w0, w1)

    def flat_base(k):
        ci, b = divmod(k, B)
        return b * S + s0 + ci * C

    def idx_ref(k):
        ci, b = divmod(k, B)
        return idx_all.at[b, pl.ds(ci * C, C)]

    def fire_gather(k, slot):
        pltpu.async_copy(word_hbm.at[idx_ref(k)], acc[slot], gsem[slot])

    def wait_gather(k, slot):
        pltpu.make_async_copy(word_hbm.at[idx_ref(k)], acc[slot],
                              gsem[slot]).wait()

    def fire_write(k, slot):
        pltpu.async_copy(acc[slot], out_hbm.at[pl.ds(flat_base(k), C)],
                         wsem[slot])

    def wait_write(k, slot):
        pltpu.make_async_copy(acc[slot], out_hbm.at[pl.ds(flat_base(k), C)],
                              wsem[slot]).wait()

    for b in range(B):
        pltpu.sync_copy(x_hbm.at[pl.ds(b * S + s0, S_PER_W)], idx_all.at[b])

    fire_gather(0, 0)

    for k in range(NITEMS):
        ci, b = divmod(k, B)
        slot = k % 2
        if b == 0:
            pltpu.sync_copy(pos_hbm.at[pl.ds(s0 + ci * C, C)], pos_v)
        if k + 1 < NITEMS:
            if k >= 1:
                wait_write(k - 1, 1 - slot)
            fire_gather(k + 1, 1 - slot)
        wait_gather(k, slot)

        @pl.loop(0, C)
        def _row(r):
            @plsc.parallel_loop(0, D, 16, unroll=8)
            def _col(c0):
                plsc.addupdate(
                    acc[slot].at[r, pl.ds(c0, 16)], pos_v[r, pl.ds(c0, 16)]
                )

        fire_write(k, slot)

    wait_write(NITEMS - 2, NITEMS % 2)
    wait_write(NITEMS - 1, 1 - NITEMS % 2)


def kernel(x, word_table, pos_table):
    out = _embed(x.reshape(B * S), word_table, pos_table)
    return out.reshape(B, S, D)


# speedup vs baseline: 2.4848x; 1.0013x over previous
import functools

import jax
import jax.numpy as jnp
from jax import lax
from jax.experimental import pallas as pl
from jax.experimental.pallas import tpu as pltpu
from jax.experimental.pallas import tpu_sc as plsc

B = 4
S = 4096
D = 2048
NC = 2
NS = 16
NW = NC * NS
S_PER_W = S // NW
C = 16
NCHUNK = S_PER_W // C
NITEMS = NCHUNK * B

_MESH = plsc.VectorSubcoreMesh(core_axis_name="c", subcore_axis_name="s")


@functools.partial(
    pl.kernel,
    out_type=jax.ShapeDtypeStruct((B * S, D), jnp.float32),
    mesh=_MESH,
    scratch_types=[
        pltpu.VMEM((B, S_PER_W), jnp.int32),
        pltpu.VMEM((C, D), jnp.float32),
        pltpu.VMEM((C, D), jnp.float32),
        pltpu.VMEM((C, D), jnp.float32),
        pltpu.SemaphoreType.DMA,
        pltpu.SemaphoreType.DMA,
        pltpu.SemaphoreType.DMA,
        pltpu.SemaphoreType.DMA,
        pltpu.SemaphoreType.DMA,
    ],
)
def _embed(x_hbm, word_hbm, pos_hbm, out_hbm,
           idx_all, pos_v, acc0, acc1, g0, g1, w0, w1, psem):
    sid = lax.axis_index("s")
    wid = sid * NC + lax.axis_index("c")
    s0 = wid * S_PER_W
    acc = (acc0, acc1)
    gsem = (g0, g1)
    wsem = (w0, w1)

    def flat_base(k):
        ci, b = divmod(k, B)
        return b * S + s0 + ci * C

    def idx_ref(k):
        ci, b = divmod(k, B)
        return idx_all.at[b, pl.ds(ci * C, C)]

    def fire_gather(k, slot):
        pltpu.async_copy(word_hbm.at[idx_ref(k)], acc[slot], gsem[slot])

    def wait_gather(k, slot):
        pltpu.make_async_copy(word_hbm.at[idx_ref(k)], acc[slot],
                              gsem[slot]).wait()

    def fire_write(k, slot):
        pltpu.async_copy(acc[slot], out_hbm.at[pl.ds(flat_base(k), C)],
                         wsem[slot])

    def wait_write(k, slot):
        pltpu.make_async_copy(acc[slot], out_hbm.at[pl.ds(flat_base(k), C)],
                              wsem[slot]).wait()

    def fire_pos(ci):
        pltpu.async_copy(pos_hbm.at[pl.ds(s0 + ci * C, C)], pos_v, psem)

    def wait_pos(ci):
        pltpu.make_async_copy(pos_hbm.at[pl.ds(s0 + ci * C, C)], pos_v,
                              psem).wait()

    for b in range(B):
        pltpu.sync_copy(x_hbm.at[pl.ds(b * S + s0, S_PER_W)], idx_all.at[b])

    fire_pos(0)
    fire_gather(0, 0)

    for k in range(NITEMS):
        ci, b = divmod(k, B)
        slot = k % 2
        if b == 0:
            wait_pos(ci)
        if k + 1 < NITEMS:
            if k >= 1:
                wait_write(k - 1, 1 - slot)
            fire_gather(k + 1, 1 - slot)
        wait_gather(k, slot)

        @pl.loop(0, C)
        def _row(r):
            @plsc.parallel_loop(0, D, 16, unroll=8)
            def _col(c0):
                plsc.addupdate(
                    acc[slot].at[r, pl.ds(c0, 16)], pos_v[r, pl.ds(c0, 16)]
                )

        if b == B - 1 and ci + 1 < NCHUNK:
            fire_pos(ci + 1)
        fire_write(k, slot)

    wait_write(NITEMS - 2, NITEMS % 2)
    wait_write(NITEMS - 1, 1 - NITEMS % 2)


def kernel(x, word_table, pos_table):
    out = _embed(x.reshape(B * S), word_table, pos_table)
    return out.reshape(B, S, D)


# speedup vs baseline: 2.8158x; 1.1332x over previous
import functools

import jax
import jax.numpy as jnp
from jax import lax
from jax.experimental import pallas as pl
from jax.experimental.pallas import tpu as pltpu
from jax.experimental.pallas import tpu_sc as plsc

B = 4
S = 4096
D = 2048
NC = 2
NS = 16
NW = NC * NS
S_PER_W = S // NW
C = 8
NCHUNK = S_PER_W // C
NITEMS = NCHUNK * B
NBUF = 4

_MESH = plsc.VectorSubcoreMesh(core_axis_name="c", subcore_axis_name="s")


@functools.partial(
    pl.kernel,
    out_type=jax.ShapeDtypeStruct((B * S, D), jnp.float32),
    mesh=_MESH,
    scratch_types=[
        pltpu.VMEM((B, S_PER_W), jnp.int32),
        [pltpu.VMEM((C, D), jnp.float32) for _ in range(2)],
        [pltpu.VMEM((C, D), jnp.float32) for _ in range(NBUF)],
        [pltpu.SemaphoreType.DMA for _ in range(NBUF)],
        [pltpu.SemaphoreType.DMA for _ in range(NBUF)],
        [pltpu.SemaphoreType.DMA for _ in range(2)],
    ],
)
def _embed(x_hbm, word_hbm, pos_hbm, out_hbm,
           idx_all, pos, acc, gsem, wsem, psem):
    wid = lax.axis_index("s") * NC + lax.axis_index("c")
    s0 = wid * S_PER_W

    def flat_base(k):
        ci, b = divmod(k, B)
        return b * S + s0 + ci * C

    def idx_ref(k):
        ci, b = divmod(k, B)
        return idx_all.at[b, pl.ds(ci * C, C)]

    def fire_gather(k, slot):
        pltpu.async_copy(word_hbm.at[idx_ref(k)], acc[slot], gsem[slot])

    def wait_gather(k, slot):
        pltpu.make_async_copy(word_hbm.at[idx_ref(k)], acc[slot],
                              gsem[slot]).wait()

    def fire_write(k, slot):
        pltpu.async_copy(acc[slot], out_hbm.at[pl.ds(flat_base(k), C)],
                         wsem[slot])

    def wait_write(k, slot):
        pltpu.make_async_copy(acc[slot], out_hbm.at[pl.ds(flat_base(k), C)],
                              wsem[slot]).wait()

    def fire_pos(ci):
        pltpu.async_copy(pos_hbm.at[pl.ds(s0 + ci * C, C)], pos[ci % 2],
                         psem[ci % 2])

    def wait_pos(ci):
        pltpu.make_async_copy(pos_hbm.at[pl.ds(s0 + ci * C, C)], pos[ci % 2],
                              psem[ci % 2]).wait()

    for b in range(B):
        pltpu.sync_copy(x_hbm.at[pl.ds(b * S + s0, S_PER_W)], idx_all.at[b])

    fire_pos(0)
    fire_gather(0, 0)

    for k in range(NITEMS):
        ci, b = divmod(k, B)
        slot = k % NBUF
        if b == 0:
            wait_pos(ci)
            if ci + 1 < NCHUNK:
                fire_pos(ci + 1)
        if k + 1 < NITEMS:
            if k + 1 >= NBUF:
                wait_write(k + 1 - NBUF, (k + 1) % NBUF)
            fire_gather(k + 1, (k + 1) % NBUF)
        wait_gather(k, slot)
        pv = pos[ci % 2]

        @pl.loop(0, C)
        def _row(r):
            @plsc.parallel_loop(0, D, 16, unroll=8)
            def _col(c0):
                plsc.addupdate(acc[slot].at[r, pl.ds(c0, 16)],
                               pv[r, pl.ds(c0, 16)])

        fire_write(k, slot)

    for j in range(max(0, NITEMS - NBUF), NITEMS):
        wait_write(j, j % NBUF)


def kernel(x, word_table, pos_table):
    out = _embed(x.reshape(B * S), word_table, pos_table)
    return out.reshape(B, S, D)
